# baseline (speedup 1.0000x reference)
import functools

import jax
import jax.numpy as jnp
from jax import lax
from jax.experimental import pallas as pl
from jax.experimental.pallas import tpu as pltpu
from jax.experimental.pallas import tpu_sc as plsc

B, S, D = 2, 2048, 1024
FF = 4096
E = 8
T = B * S
CAP = T // E
TB = 1024
NTB = T // TB
FFB = 1024
NFFB = FF // FFB

NC, NS = 2, 16
NW = NC * NS
RPT = T // NW
GCH = 32
NCH = RPT // GCH


def _pipelined_gather(src_hbm, idx_ref, dst_hbm, dst_base, rows_v,
                      gsems, osems):
    gets = [None] * NCH
    puts = [None] * NCH
    for c in range(NCH):
        if c >= 2:
            puts[c - 2].wait()
        gets[c] = pltpu.async_copy(
            src_hbm.at[idx_ref.at[pl.ds(c * GCH, GCH)]],
            rows_v.at[c % 2], gsems[c % 2])
        if c >= 1:
            gets[c - 1].wait()
            puts[c - 1] = pltpu.async_copy(
                rows_v.at[(c - 1) % 2],
                dst_hbm.at[pl.ds(dst_base + (c - 1) * GCH, GCH), :],
                osems[(c - 1) % 2])
    gets[NCH - 1].wait()
    puts[NCH - 1] = pltpu.async_copy(
        rows_v.at[(NCH - 1) % 2],
        dst_hbm.at[pl.ds(dst_base + (NCH - 1) * GCH, GCH), :],
        osems[(NCH - 1) % 2])
    if NCH >= 2:
        puts[NCH - 2].wait()
    puts[NCH - 1].wait()


def _router_body(x_ref, wr_ref, slot_ref, prob_ref, carry_ref):
    pid = pl.program_id(0)

    @pl.when(pid == 0)
    def _():
        carry_ref[...] = jnp.zeros_like(carry_ref)

    xb = x_ref[...]
    logits = lax.dot_general(wr_ref[...], xb,
                             (((1,), (1,)), ((), ())),
                             preferred_element_type=jnp.float32)
    lmax = jnp.max(logits, axis=0, keepdims=True)
    ssum = jnp.sum(jnp.exp(logits - lmax), axis=0)
    iota_e = lax.broadcasted_iota(jnp.int32, (E, TB), 0)
    is_max = logits == lmax
    idx = jnp.min(jnp.where(is_max, iota_e, E), axis=0)
    onehot = (iota_e == idx[None, :])

    r_io = lax.broadcasted_iota(jnp.int32, (TB, TB), 0)
    c_io = lax.broadcasted_iota(jnp.int32, (TB, TB), 1)
    utri = (r_io <= c_io).astype(jnp.bfloat16)
    csum = lax.dot_general(onehot.astype(jnp.bfloat16), utri,
                           (((1,), (0,)), ((), ())),
                           preferred_element_type=jnp.float32)
    ohf = onehot.astype(jnp.float32)
    rank_in_blk = jnp.sum(csum * ohf, axis=0) - 1.0
    carry = carry_ref[...]
    base = jnp.sum(carry * ohf, axis=0)
    rank = (rank_in_blk + base).astype(jnp.int32)
    carry_ref[...] = carry + jnp.sum(ohf, axis=1, keepdims=True)

    slot = jnp.where(rank < CAP, idx * CAP + rank, E * CAP)
    slot_ref[...] = slot.reshape(1, 1, TB)
    prob_ref[...] = (1.0 / ssum).reshape(1, 1, TB)


def _router(x_flat, Wr):
    slot, prob = pl.pallas_call(
        _router_body,
        grid=(NTB,),
        in_specs=[
            pl.BlockSpec((TB, D), lambda i: (i, 0)),
            pl.BlockSpec((E, D), lambda i: (0, 0)),
        ],
        out_specs=[
            pl.BlockSpec((1, 1, TB), lambda i: (i, 0, 0)),
            pl.BlockSpec((1, 1, TB), lambda i: (i, 0, 0)),
        ],
        out_shape=[
            jax.ShapeDtypeStruct((NTB, 1, TB), jnp.int32),
            jax.ShapeDtypeStruct((NTB, 1, TB), jnp.float32),
        ],
        scratch_shapes=[pltpu.VMEM((E, 1), jnp.float32)],
    )(x_flat, Wr)
    return slot.reshape(T), prob.reshape(T)


CH_E = 2
NCHK = E // CH_E
CROWS = CH_E * CAP
RPT_C = CROWS // NW


def _invbuild_body(slot_hbm, prob_hbm, ids_hbm, pslot_hbm,
                   slot_v, prob_v, ids_v, ps_v):
    wid = lax.axis_index("s") * NC + lax.axis_index("c")
    base = wid * RPT

    pltpu.sync_copy(slot_hbm, slot_v)
    pltpu.sync_copy(prob_hbm, prob_v)

    zero16 = jnp.zeros((16,), jnp.int32)

    def init_body(c, _):
        ids_v[pl.ds(c * 16, 16)] = zero16
        return 0

    lax.fori_loop(0, T // 16, init_body, 0)

    i16 = lax.iota(jnp.int32, 16)

    def scat_body(c, _):
        sv = slot_v[pl.ds(c * 16, 16)]
        m = sv < T
        plsc.store_scatter(ids_v, [sv], i16 + c * 16, mask=m)
        plsc.store_scatter(ps_v, [sv], prob_v[pl.ds(c * 16, 16)], mask=m)
        return 0

    lax.fori_loop(0, T // 16, scat_body, 0)

    pltpu.sync_copy(ids_v.at[pl.ds(base, RPT)], ids_hbm.at[pl.ds(base, RPT)])
    pltpu.sync_copy(ps_v.at[pl.ds(base, RPT)], pslot_hbm.at[pl.ds(base, RPT)])


def _invbuild(slot, prob):
    mesh = plsc.VectorSubcoreMesh(core_axis_name="c", subcore_axis_name="s")
    k = pl.kernel(
        _invbuild_body,
        compiler_params=pltpu.CompilerParams(needs_layout_passes=False),
        out_type=[
            jax.ShapeDtypeStruct((T,), jnp.int32),
            jax.ShapeDtypeStruct((T,), jnp.float32),
        ],
        mesh=mesh,
        scratch_types=[
            pltpu.VMEM((T,), jnp.int32),
            pltpu.VMEM((T,), jnp.float32),
            pltpu.VMEM((T,), jnp.int32),
            pltpu.VMEM((T,), jnp.float32),
        ],
    )
    return k(slot, prob)


def _make_gather_chunk(koff):
    def body(x_hbm, ids_hbm, xe_hbm, ids_v, rows_v, gsem, osem):
        wid = lax.axis_index("s") * NC + lax.axis_index("c")
        base = koff + wid * RPT_C
        pltpu.sync_copy(ids_hbm.at[pl.ds(base, RPT_C)], ids_v)
        pltpu.async_copy(x_hbm.at[ids_v], rows_v, gsem).wait()
        pltpu.async_copy(rows_v, xe_hbm.at[pl.ds(wid * RPT_C, RPT_C), :],
                         osem).wait()

    mesh = plsc.VectorSubcoreMesh(core_axis_name="c", subcore_axis_name="s")
    return pl.kernel(
        body,
        compiler_params=pltpu.CompilerParams(needs_layout_passes=False),
        out_type=jax.ShapeDtypeStruct((CROWS, D), jnp.float32),
        mesh=mesh,
        scratch_types=[
            pltpu.VMEM((RPT_C,), jnp.int32),
            pltpu.VMEM((RPT_C, D), jnp.float32),
            pltpu.SemaphoreType.DMA,
            pltpu.SemaphoreType.DMA,
        ],
    )


def _ffn_body(x_ref, w1_ref, w2_ref, p_ref, res_ref, out_ref, acc_ref):
    f = pl.program_id(1)

    xb = x_ref[0].astype(jnp.bfloat16)
    w1 = w1_ref[0].astype(jnp.bfloat16)
    h = lax.dot_general(xb, w1, (((1,), (1,)), ((), ())),
                        preferred_element_type=jnp.float32)
    h = jnp.maximum(h, 0.0).astype(jnp.bfloat16)
    w2 = w2_ref[0].astype(jnp.bfloat16)
    part = lax.dot_general(h, w2, (((1,), (1,)), ((), ())),
                           preferred_element_type=jnp.float32)

    @pl.when(f == 0)
    def _():
        acc_ref[...] = part

    @pl.when(f > 0)
    def _():
        acc_ref[...] = acc_ref[...] + part

    @pl.when(f == NFFB - 1)
    def _():
        pv = p_ref[0, 0][:, None]
        out_ref[0] = acc_ref[...] * pv


def _ffn_chunk(xk, W1, W2, pk, res, k):
    xk3 = xk.reshape(CH_E, CAP, D)
    pk3 = pk.reshape(CH_E, 1, CAP)
    return pl.pallas_call(
        _ffn_body,
        grid=(CH_E, NFFB),
        in_specs=[
            pl.BlockSpec((1, CAP, D), lambda e, f: (e, 0, 0)),
            pl.BlockSpec((1, FFB, D), lambda e, f: (CH_E * k + e, f, 0)),
            pl.BlockSpec((1, D, FFB), lambda e, f: (CH_E * k + e, 0, f)),
            pl.BlockSpec((1, 1, CAP), lambda e, f: (e, 0, 0)),
            pl.BlockSpec((1, CAP, D), lambda e, f: (CH_E * k + e, 0, 0)),
        ],
        out_specs=pl.BlockSpec((1, CAP, D), lambda e, f: (CH_E * k + e, 0, 0)),
        out_shape=jax.ShapeDtypeStruct((E + 1, CAP, D), jnp.float32),
        input_output_aliases={4: 0},
        scratch_shapes=[pltpu.VMEM((CAP, D), jnp.float32)],
    )(xk3, W1, W2, pk3, res)


def _collect_body(res_hbm, slot_hbm, y_hbm, slot_v, rows_v,
                  gs0, gs1, os0, os1):
    wid = lax.axis_index("s") * NC + lax.axis_index("c")
    base = wid * RPT

    pltpu.sync_copy(slot_hbm.at[pl.ds(base, RPT)], slot_v)

    _pipelined_gather(res_hbm, slot_v, y_hbm, base,
                      rows_v, (gs0, gs1), (os0, os1))


def _collect(result, slot):
    mesh = plsc.VectorSubcoreMesh(core_axis_name="c", subcore_axis_name="s")
    k = pl.kernel(
        _collect_body,
        compiler_params=pltpu.CompilerParams(needs_layout_passes=False),
        out_type=jax.ShapeDtypeStruct((T, D), jnp.float32),
        mesh=mesh,
        scratch_types=[
            pltpu.VMEM((RPT,), jnp.int32),
            pltpu.VMEM((2, GCH, D), jnp.float32),
            pltpu.SemaphoreType.DMA,
            pltpu.SemaphoreType.DMA,
            pltpu.SemaphoreType.DMA,
            pltpu.SemaphoreType.DMA,
        ],
    )
    return k(result, slot)


@jax.jit
def kernel(x, Wr, W1, W2):
    x_flat = x.reshape(T, D)
    slot, prob = _router(x_flat, Wr)
    ids, pslot = _invbuild(slot, prob)
    res = jnp.zeros((E + 1, CAP, D), jnp.float32)
    for k in range(NCHK):
        xk = _make_gather_chunk(k * CROWS)(x_flat, ids)
        pk = lax.dynamic_slice_in_dim(pslot, k * CROWS, CROWS)
        res = _ffn_chunk(xk, W1, W2, pk, res, k)
    y = _collect(res.reshape((E + 1) * CAP, D), slot)
    return y.reshape(B, S, D)


# speedup vs baseline: 1.1762x; 1.1762x over previous
import functools

import jax
import jax.numpy as jnp
from jax import lax
from jax.experimental import pallas as pl
from jax.experimental.pallas import tpu as pltpu
from jax.experimental.pallas import tpu_sc as plsc

B, S, D = 2, 2048, 1024
FF = 4096
E = 8
T = B * S
CAP = T // E
TB = 1024
NTB = T // TB
FFB = 1024
NFFB = FF // FFB

NC, NS = 2, 16
NW = NC * NS
RPT = T // NW
GCH = 32
NCH = RPT // GCH


def _pipelined_gather(src_hbm, idx_ref, dst_hbm, dst_base, rows_v,
                      gsems, osems):
    gets = [None] * NCH
    puts = [None] * NCH
    for c in range(NCH):
        if c >= 2:
            puts[c - 2].wait()
        gets[c] = pltpu.async_copy(
            src_hbm.at[idx_ref.at[pl.ds(c * GCH, GCH)]],
            rows_v.at[c % 2], gsems[c % 2])
        if c >= 1:
            gets[c - 1].wait()
            puts[c - 1] = pltpu.async_copy(
                rows_v.at[(c - 1) % 2],
                dst_hbm.at[pl.ds(dst_base + (c - 1) * GCH, GCH), :],
                osems[(c - 1) % 2])
    gets[NCH - 1].wait()
    puts[NCH - 1] = pltpu.async_copy(
        rows_v.at[(NCH - 1) % 2],
        dst_hbm.at[pl.ds(dst_base + (NCH - 1) * GCH, GCH), :],
        osems[(NCH - 1) % 2])
    if NCH >= 2:
        puts[NCH - 2].wait()
    puts[NCH - 1].wait()


def _pipelined_scatter(src_hbm, src_base, idx_ref, dst_hbm, rows_v,
                       gsems, osems):
    gets = [None] * NCH
    puts = [None] * NCH
    for c in range(NCH):
        if c >= 2:
            puts[c - 2].wait()
        gets[c] = pltpu.async_copy(
            src_hbm.at[pl.ds(src_base + c * GCH, GCH), :],
            rows_v.at[c % 2], gsems[c % 2])
        if c >= 1:
            gets[c - 1].wait()
            puts[c - 1] = pltpu.async_copy(
                rows_v.at[(c - 1) % 2],
                dst_hbm.at[idx_ref.at[pl.ds((c - 1) * GCH, GCH)]],
                osems[(c - 1) % 2])
    gets[NCH - 1].wait()
    puts[NCH - 1] = pltpu.async_copy(
        rows_v.at[(NCH - 1) % 2],
        dst_hbm.at[idx_ref.at[pl.ds((NCH - 1) * GCH, GCH)]],
        osems[(NCH - 1) % 2])
    if NCH >= 2:
        puts[NCH - 2].wait()
    puts[NCH - 1].wait()


def _router_body(x_ref, wr_ref, slot_ref, xs_ref, carry_ref):
    pid = pl.program_id(0)

    @pl.when(pid == 0)
    def _():
        carry_ref[...] = jnp.zeros_like(carry_ref)

    xb = x_ref[...]
    logits = lax.dot_general(wr_ref[...], xb,
                             (((1,), (1,)), ((), ())),
                             preferred_element_type=jnp.float32)
    lmax = jnp.max(logits, axis=0, keepdims=True)
    ssum = jnp.sum(jnp.exp(logits - lmax), axis=0)
    iota_e = lax.broadcasted_iota(jnp.int32, (E, TB), 0)
    is_max = logits == lmax
    idx = jnp.min(jnp.where(is_max, iota_e, E), axis=0)
    onehot = (iota_e == idx[None, :])

    r_io = lax.broadcasted_iota(jnp.int32, (TB, TB), 0)
    c_io = lax.broadcasted_iota(jnp.int32, (TB, TB), 1)
    utri = (r_io <= c_io).astype(jnp.bfloat16)
    csum = lax.dot_general(onehot.astype(jnp.bfloat16), utri,
                           (((1,), (0,)), ((), ())),
                           preferred_element_type=jnp.float32)
    ohf = onehot.astype(jnp.float32)
    rank_in_blk = jnp.sum(csum * ohf, axis=0) - 1.0
    carry = carry_ref[...]
    base = jnp.sum(carry * ohf, axis=0)
    rank = (rank_in_blk + base).astype(jnp.int32)
    carry_ref[...] = carry + jnp.sum(ohf, axis=1, keepdims=True)

    slot = jnp.where(rank < CAP, idx * CAP + rank, E * CAP)
    slot_ref[...] = slot.reshape(1, 1, TB)
    xs_ref[...] = xb * (1.0 / ssum)[:, None]


def _router(x_flat, Wr):
    slot, xs = pl.pallas_call(
        _router_body,
        grid=(NTB,),
        in_specs=[
            pl.BlockSpec((TB, D), lambda i: (i, 0)),
            pl.BlockSpec((E, D), lambda i: (0, 0)),
        ],
        out_specs=[
            pl.BlockSpec((1, 1, TB), lambda i: (i, 0, 0)),
            pl.BlockSpec((TB, D), lambda i: (i, 0)),
        ],
        out_shape=[
            jax.ShapeDtypeStruct((NTB, 1, TB), jnp.int32),
            jax.ShapeDtypeStruct((T, D), jnp.float32),
        ],
        scratch_shapes=[pltpu.VMEM((E, 1), jnp.float32)],
    )(x_flat, Wr)
    return slot.reshape(T), xs


def _dispatch_body(xs_hbm, slot_hbm, xe_hbm,
                   slot_v, rows_v, gs0, gs1, os0, os1):
    wid = lax.axis_index("s") * NC + lax.axis_index("c")
    base = wid * RPT

    pltpu.sync_copy(slot_hbm.at[pl.ds(base, RPT)], slot_v)

    _pipelined_scatter(xs_hbm, base, slot_v, xe_hbm, rows_v,
                       (gs0, gs1), (os0, os1))


def _dispatch(xs, slot):
    mesh = plsc.VectorSubcoreMesh(core_axis_name="c", subcore_axis_name="s")
    k = pl.kernel(
        _dispatch_body,
        compiler_params=pltpu.CompilerParams(needs_layout_passes=False),
        out_type=jax.ShapeDtypeStruct((T + CAP, D), jnp.float32),
        mesh=mesh,
        scratch_types=[
            pltpu.VMEM((RPT,), jnp.int32),
            pltpu.VMEM((2, GCH, D), jnp.float32),
            pltpu.SemaphoreType.DMA,
            pltpu.SemaphoreType.DMA,
            pltpu.SemaphoreType.DMA,
            pltpu.SemaphoreType.DMA,
        ],
    )
    return k(xs, slot)


def _ffn_body(x_ref, w1_ref, w2_ref, out_ref, acc_ref):
    e = pl.program_id(0)
    f = pl.program_id(1)

    @pl.when(e < E)
    def _():
        xb = x_ref[0].astype(jnp.bfloat16)
        w1 = w1_ref[0].astype(jnp.bfloat16)
        h = lax.dot_general(xb, w1, (((1,), (1,)), ((), ())),
                            preferred_element_type=jnp.float32)
        h = jnp.maximum(h, 0.0).astype(jnp.bfloat16)
        w2 = w2_ref[0].astype(jnp.bfloat16)
        part = lax.dot_general(h, w2, (((1,), (1,)), ((), ())),
                               preferred_element_type=jnp.float32)

        @pl.when(f == 0)
        def _():
            acc_ref[...] = part

        @pl.when(f > 0)
        def _():
            acc_ref[...] = acc_ref[...] + part

        @pl.when(f == NFFB - 1)
        def _():
            out_ref[0] = acc_ref[...]

    @pl.when(e == E)
    def _():
        out_ref[0] = jnp.zeros_like(out_ref[0])


def _ffn(xe, W1, W2):
    xe3 = xe.reshape(E + 1, CAP, D)
    clamp_e = lambda e: jnp.minimum(e, E - 1)
    wf = lambda e, f: jnp.where(e < E, f, NFFB - 1)
    out = pl.pallas_call(
        _ffn_body,
        grid=(E + 1, NFFB),
        in_specs=[
            pl.BlockSpec((1, CAP, D), lambda e, f: (clamp_e(e), 0, 0)),
            pl.BlockSpec((1, FFB, D), lambda e, f: (clamp_e(e), wf(e, f), 0)),
            pl.BlockSpec((1, D, FFB), lambda e, f: (clamp_e(e), 0, wf(e, f))),
        ],
        out_specs=pl.BlockSpec((1, CAP, D), lambda e, f: (e, 0, 0)),
        out_shape=jax.ShapeDtypeStruct((E + 1, CAP, D), jnp.float32),
        scratch_shapes=[pltpu.VMEM((CAP, D), jnp.float32)],
    )(xe3, W1, W2)
    return out.reshape((E + 1) * CAP, D)


def _collect_body(res_hbm, slot_hbm, y_hbm, slot_v, rows_v,
                  gs0, gs1, os0, os1):
    wid = lax.axis_index("s") * NC + lax.axis_index("c")
    base = wid * RPT

    pltpu.sync_copy(slot_hbm.at[pl.ds(base, RPT)], slot_v)

    _pipelined_gather(res_hbm, slot_v, y_hbm, base,
                      rows_v, (gs0, gs1), (os0, os1))


def _collect(result, slot):
    mesh = plsc.VectorSubcoreMesh(core_axis_name="c", subcore_axis_name="s")
    k = pl.kernel(
        _collect_body,
        compiler_params=pltpu.CompilerParams(needs_layout_passes=False),
        out_type=jax.ShapeDtypeStruct((T, D), jnp.float32),
        mesh=mesh,
        scratch_types=[
            pltpu.VMEM((RPT,), jnp.int32),
            pltpu.VMEM((2, GCH, D), jnp.float32),
            pltpu.SemaphoreType.DMA,
            pltpu.SemaphoreType.DMA,
            pltpu.SemaphoreType.DMA,
            pltpu.SemaphoreType.DMA,
        ],
    )
    return k(result, slot)


@jax.jit
def kernel(x, Wr, W1, W2):
    x_flat = x.reshape(T, D)
    slot, xs = _router(x_flat, Wr)
    xe = _dispatch(xs, slot)
    result = _ffn(xe, W1, W2)
    y = _collect(result, slot)
    return y.reshape(B, S, D)


# speedup vs baseline: 1.2310x; 1.0466x over previous
import functools

import jax
import jax.numpy as jnp
from jax import lax
from jax.experimental import pallas as pl
from jax.experimental.pallas import tpu as pltpu
from jax.experimental.pallas import tpu_sc as plsc

B, S, D = 2, 2048, 1024
FF = 4096
E = 8
T = B * S
CAP = T // E
TB = 1024
NTB = T // TB
FFB = 1024
NFFB = FF // FFB

NC, NS = 2, 16
NW = NC * NS
RPT = T // NW
GCH = 32
NCH = RPT // GCH


def _pipelined_gather(src_hbm, idx_ref, dst_hbm, dst_base, rows_v,
                      gsems, osems):
    gets = [None] * NCH
    puts = [None] * NCH
    for c in range(NCH):
        if c >= 2:
            puts[c - 2].wait()
        gets[c] = pltpu.async_copy(
            src_hbm.at[idx_ref.at[pl.ds(c * GCH, GCH)]],
            rows_v.at[c % 2], gsems[c % 2])
        if c >= 1:
            gets[c - 1].wait()
            puts[c - 1] = pltpu.async_copy(
                rows_v.at[(c - 1) % 2],
                dst_hbm.at[pl.ds(dst_base + (c - 1) * GCH, GCH), :],
                osems[(c - 1) % 2])
    gets[NCH - 1].wait()
    puts[NCH - 1] = pltpu.async_copy(
        rows_v.at[(NCH - 1) % 2],
        dst_hbm.at[pl.ds(dst_base + (NCH - 1) * GCH, GCH), :],
        osems[(NCH - 1) % 2])
    if NCH >= 2:
        puts[NCH - 2].wait()
    puts[NCH - 1].wait()


def _pipelined_scatter(src_hbm, src_base, idx_ref, dst_hbm, rows_v,
                       gsems, osems):
    gets = [None] * NCH
    puts = [None] * NCH
    for c in range(NCH):
        if c >= 2:
            puts[c - 2].wait()
        gets[c] = pltpu.async_copy(
            src_hbm.at[pl.ds(src_base + c * GCH, GCH), :],
            rows_v.at[c % 2], gsems[c % 2])
        if c >= 1:
            gets[c - 1].wait()
            puts[c - 1] = pltpu.async_copy(
                rows_v.at[(c - 1) % 2],
                dst_hbm.at[idx_ref.at[pl.ds((c - 1) * GCH, GCH)]],
                osems[(c - 1) % 2])
    gets[NCH - 1].wait()
    puts[NCH - 1] = pltpu.async_copy(
        rows_v.at[(NCH - 1) % 2],
        dst_hbm.at[idx_ref.at[pl.ds((NCH - 1) * GCH, GCH)]],
        osems[(NCH - 1) % 2])
    if NCH >= 2:
        puts[NCH - 2].wait()
    puts[NCH - 1].wait()


def _router_body(x_ref, wr_ref, slot_ref, xs_ref, carry_ref):
    pid = pl.program_id(0)

    @pl.when(pid == 0)
    def _():
        carry_ref[...] = jnp.zeros_like(carry_ref)

    xb = x_ref[...]
    logits = lax.dot_general(wr_ref[...], xb,
                             (((1,), (1,)), ((), ())),
                             preferred_element_type=jnp.float32)
    lmax = jnp.max(logits, axis=0, keepdims=True)
    ssum = jnp.sum(jnp.exp(logits - lmax), axis=0)
    iota_e = lax.broadcasted_iota(jnp.int32, (E, TB), 0)
    is_max = logits == lmax
    idx = jnp.min(jnp.where(is_max, iota_e, E), axis=0)
    onehot = (iota_e == idx[None, :])

    r_io = lax.broadcasted_iota(jnp.int32, (TB, TB), 0)
    c_io = lax.broadcasted_iota(jnp.int32, (TB, TB), 1)
    utri = (r_io <= c_io).astype(jnp.bfloat16)
    csum = lax.dot_general(onehot.astype(jnp.bfloat16), utri,
                           (((1,), (0,)), ((), ())),
                           preferred_element_type=jnp.float32)
    ohf = onehot.astype(jnp.float32)
    rank_in_blk = jnp.sum(csum * ohf, axis=0) - 1.0
    carry = carry_ref[...]
    base = jnp.sum(carry * ohf, axis=0)
    rank = (rank_in_blk + base).astype(jnp.int32)
    carry_ref[...] = carry + jnp.sum(ohf, axis=1, keepdims=True)

    slot = jnp.where(rank < CAP, idx * CAP + rank, E * CAP)
    slot_ref[...] = slot.reshape(1, 1, TB)

    xsc = xb * (1.0 / ssum)[:, None]
    bits = lax.bitcast_convert_type(xsc, jnp.int32)
    rnd = (bits + 0x7FFF + ((bits >> 16) & 1)) >> 16
    lo = rnd[:, :D // 2] & 0xFFFF
    hi = rnd[:, D // 2:] << 16
    xs_ref[...] = lax.bitcast_convert_type(hi | lo, jnp.float32)


def _router(x_flat, Wr):
    slot, xs = pl.pallas_call(
        _router_body,
        grid=(NTB,),
        in_specs=[
            pl.BlockSpec((TB, D), lambda i: (i, 0)),
            pl.BlockSpec((E, D), lambda i: (0, 0)),
        ],
        out_specs=[
            pl.BlockSpec((1, 1, TB), lambda i: (i, 0, 0)),
            pl.BlockSpec((TB, D // 2), lambda i: (i, 0)),
        ],
        out_shape=[
            jax.ShapeDtypeStruct((NTB, 1, TB), jnp.int32),
            jax.ShapeDtypeStruct((T, D // 2), jnp.float32),
        ],
        scratch_shapes=[pltpu.VMEM((E, 1), jnp.float32)],
    )(x_flat, Wr)
    return slot.reshape(T), xs


def _dispatch_body(xs_hbm, slot_hbm, xe_hbm,
                   slot_v, rows_v, gs0, gs1, os0, os1):
    wid = lax.axis_index("s") * NC + lax.axis_index("c")
    base = wid * RPT

    pltpu.sync_copy(slot_hbm.at[pl.ds(base, RPT)], slot_v)

    _pipelined_scatter(xs_hbm, base, slot_v, xe_hbm, rows_v,
                       (gs0, gs1), (os0, os1))


def _dispatch(xs, slot):
    mesh = plsc.VectorSubcoreMesh(core_axis_name="c", subcore_axis_name="s")
    k = pl.kernel(
        _dispatch_body,
        compiler_params=pltpu.CompilerParams(needs_layout_passes=False),
        out_type=jax.ShapeDtypeStruct((T + CAP, D // 2), jnp.float32),
        mesh=mesh,
        scratch_types=[
            pltpu.VMEM((RPT,), jnp.int32),
            pltpu.VMEM((2, GCH, D // 2), jnp.float32),
            pltpu.SemaphoreType.DMA,
            pltpu.SemaphoreType.DMA,
            pltpu.SemaphoreType.DMA,
            pltpu.SemaphoreType.DMA,
        ],
    )
    return k(xs, slot)


def _ffn_body(x_ref, w1_ref, w2_ref, out_ref, acc_ref):
    e = pl.program_id(0)
    f = pl.program_id(1)

    @pl.when(e < E)
    def _():
        ai = lax.bitcast_convert_type(x_ref[0], jnp.int32)
        xl = lax.bitcast_convert_type(ai << 16, jnp.float32)
        xh = lax.bitcast_convert_type(ai & jnp.int32(-65536), jnp.float32)
        xl = xl.astype(jnp.bfloat16)
        xh = xh.astype(jnp.bfloat16)
        w1 = w1_ref[0].astype(jnp.bfloat16)
        h = lax.dot_general(xl, w1[:, :D // 2], (((1,), (1,)), ((), ())),
                            preferred_element_type=jnp.float32)
        h = h + lax.dot_general(xh, w1[:, D // 2:], (((1,), (1,)), ((), ())),
                                preferred_element_type=jnp.float32)
        h = jnp.maximum(h, 0.0).astype(jnp.bfloat16)
        w2 = w2_ref[0].astype(jnp.bfloat16)
        part = lax.dot_general(h, w2, (((1,), (1,)), ((), ())),
                               preferred_element_type=jnp.float32)

        @pl.when(f == 0)
        def _():
            acc_ref[...] = part

        @pl.when(f > 0)
        def _():
            acc_ref[...] = acc_ref[...] + part

        @pl.when(f == NFFB - 1)
        def _():
            out_ref[0] = acc_ref[...]

    @pl.when(e == E)
    def _():
        out_ref[0] = jnp.zeros_like(out_ref[0])


def _ffn(xe, W1, W2):
    xe3 = xe.reshape(E + 1, CAP, D // 2)
    clamp_e = lambda e: jnp.minimum(e, E - 1)
    wf = lambda e, f: jnp.where(e < E, f, NFFB - 1)
    out = pl.pallas_call(
        _ffn_body,
        grid=(E + 1, NFFB),
        in_specs=[
            pl.BlockSpec((1, CAP, D // 2), lambda e, f: (clamp_e(e), 0, 0)),
            pl.BlockSpec((1, FFB, D), lambda e, f: (clamp_e(e), wf(e, f), 0)),
            pl.BlockSpec((1, D, FFB), lambda e, f: (clamp_e(e), 0, wf(e, f))),
        ],
        out_specs=pl.BlockSpec((1, CAP, D), lambda e, f: (e, 0, 0)),
        out_shape=jax.ShapeDtypeStruct((E + 1, CAP, D), jnp.float32),
        scratch_shapes=[pltpu.VMEM((CAP, D), jnp.float32)],
    )(xe3, W1, W2)
    return out.reshape((E + 1) * CAP, D)


def _collect_body(res_hbm, slot_hbm, y_hbm, slot_v, rows_v,
                  gs0, gs1, os0, os1):
    wid = lax.axis_index("s") * NC + lax.axis_index("c")
    base = wid * RPT

    pltpu.sync_copy(slot_hbm.at[pl.ds(base, RPT)], slot_v)

    _pipelined_gather(res_hbm, slot_v, y_hbm, base,
                      rows_v, (gs0, gs1), (os0, os1))


def _collect(result, slot):
    mesh = plsc.VectorSubcoreMesh(core_axis_name="c", subcore_axis_name="s")
    k = pl.kernel(
        _collect_body,
        compiler_params=pltpu.CompilerParams(needs_layout_passes=False),
        out_type=jax.ShapeDtypeStruct((T, D), jnp.float32),
        mesh=mesh,
        scratch_types=[
            pltpu.VMEM((RPT,), jnp.int32),
            pltpu.VMEM((2, GCH, D), jnp.float32),
            pltpu.SemaphoreType.DMA,
            pltpu.SemaphoreType.DMA,
            pltpu.SemaphoreType.DMA,
            pltpu.SemaphoreType.DMA,
        ],
    )
    return k(result, slot)


@jax.jit
def kernel(x, Wr, W1, W2):
    x_flat = x.reshape(T, D)
    slot, xs = _router(x_flat, Wr)
    xe = _dispatch(xs, slot)
    result = _ffn(xe, W1, W2)
    y = _collect(result, slot)
    return y.reshape(B, S, D)


# speedup vs baseline: 1.2391x; 1.0066x over previous
import functools

import jax
import jax.numpy as jnp
from jax import lax
from jax.experimental import pallas as pl
from jax.experimental.pallas import tpu as pltpu
from jax.experimental.pallas import tpu_sc as plsc

B, S, D = 2, 2048, 1024
FF = 4096
E = 8
T = B * S
CAP = T // E
TB = 1024
NTB = T // TB
FFB = 1024
NFFB = FF // FFB

NC, NS = 2, 16
NW = NC * NS
RPT = T // NW
GCH = 32
NCH = RPT // GCH


def _pipelined_gather(src_hbm, idx_ref, dst_hbm, dst_base, rows_v,
                      gsems, osems):
    gets = [None] * NCH
    puts = [None] * NCH
    for c in range(NCH):
        if c >= 2:
            puts[c - 2].wait()
        gets[c] = pltpu.async_copy(
            src_hbm.at[idx_ref.at[pl.ds(c * GCH, GCH)]],
            rows_v.at[c % 2], gsems[c % 2])
        if c >= 1:
            gets[c - 1].wait()
            puts[c - 1] = pltpu.async_copy(
                rows_v.at[(c - 1) % 2],
                dst_hbm.at[pl.ds(dst_base + (c - 1) * GCH, GCH), :],
                osems[(c - 1) % 2])
    gets[NCH - 1].wait()
    puts[NCH - 1] = pltpu.async_copy(
        rows_v.at[(NCH - 1) % 2],
        dst_hbm.at[pl.ds(dst_base + (NCH - 1) * GCH, GCH), :],
        osems[(NCH - 1) % 2])
    if NCH >= 2:
        puts[NCH - 2].wait()
    puts[NCH - 1].wait()


def _router_body(x_ref, wr_ref, slot_ref, xs_ref, carry_ref):
    pid = pl.program_id(0)

    @pl.when(pid == 0)
    def _():
        carry_ref[...] = jnp.zeros_like(carry_ref)

    xb = x_ref[...]
    logits = lax.dot_general(wr_ref[...], xb,
                             (((1,), (1,)), ((), ())),
                             preferred_element_type=jnp.float32)
    lmax = jnp.max(logits, axis=0, keepdims=True)
    ssum = jnp.sum(jnp.exp(logits - lmax), axis=0)
    iota_e = lax.broadcasted_iota(jnp.int32, (E, TB), 0)
    is_max = logits == lmax
    idx = jnp.min(jnp.where(is_max, iota_e, E), axis=0)
    onehot = (iota_e == idx[None, :])

    r_io = lax.broadcasted_iota(jnp.int32, (TB, TB), 0)
    c_io = lax.broadcasted_iota(jnp.int32, (TB, TB), 1)
    utri = (r_io <= c_io).astype(jnp.bfloat16)
    csum = lax.dot_general(onehot.astype(jnp.bfloat16), utri,
                           (((1,), (0,)), ((), ())),
                           preferred_element_type=jnp.float32)
    ohf = onehot.astype(jnp.float32)
    rank_in_blk = jnp.sum(csum * ohf, axis=0) - 1.0
    carry = carry_ref[...]
    base = jnp.sum(carry * ohf, axis=0)
    rank = (rank_in_blk + base).astype(jnp.int32)
    carry_ref[...] = carry + jnp.sum(ohf, axis=1, keepdims=True)

    slot = jnp.where(rank < CAP, idx * CAP + rank, E * CAP)
    slot_ref[...] = slot.reshape(1, 1, TB)

    xsc = xb * (1.0 / ssum)[:, None]
    bits = lax.bitcast_convert_type(xsc, jnp.int32)
    rnd = (bits + 0x7FFF + ((bits >> 16) & 1)) >> 16
    lo = rnd[:, :D // 2] & 0xFFFF
    hi = rnd[:, D // 2:] << 16
    xs_ref[...] = lax.bitcast_convert_type(hi | lo, jnp.float32)


def _router(x_flat, Wr):
    slot, xs = pl.pallas_call(
        _router_body,
        grid=(NTB,),
        in_specs=[
            pl.BlockSpec((TB, D), lambda i: (i, 0)),
            pl.BlockSpec((E, D), lambda i: (0, 0)),
        ],
        out_specs=[
            pl.BlockSpec((1, 1, TB), lambda i: (i, 0, 0)),
            pl.BlockSpec((TB, D // 2), lambda i: (i, 0)),
        ],
        out_shape=[
            jax.ShapeDtypeStruct((NTB, 1, TB), jnp.int32),
            jax.ShapeDtypeStruct((T, D // 2), jnp.float32),
        ],
        scratch_shapes=[pltpu.VMEM((E, 1), jnp.float32)],
    )(x_flat, Wr)
    return slot.reshape(T), xs


def _dispatch_body(xs_hbm, slot_hbm, xe_hbm,
                   slot_v, rows_v, gsem, osem):
    wid = lax.axis_index("s") * NC + lax.axis_index("c")
    base = wid * RPT

    cp_in = pltpu.async_copy(
        xs_hbm.at[pl.ds(base, RPT), :], rows_v, gsem)
    pltpu.sync_copy(slot_hbm.at[pl.ds(base, RPT)], slot_v)
    cp_in.wait()
    cp_out = pltpu.async_copy(
        rows_v, xe_hbm.at[slot_v.at[pl.ds(0, RPT)]], osem)
    cp_out.wait()


def _dispatch(xs, slot):
    mesh = plsc.VectorSubcoreMesh(core_axis_name="c", subcore_axis_name="s")
    k = pl.kernel(
        _dispatch_body,
        compiler_params=pltpu.CompilerParams(needs_layout_passes=False),
        out_type=jax.ShapeDtypeStruct((T + CAP, D // 2), jnp.float32),
        mesh=mesh,
        scratch_types=[
            pltpu.VMEM((RPT,), jnp.int32),
            pltpu.VMEM((RPT, D // 2), jnp.float32),
            pltpu.SemaphoreType.DMA,
            pltpu.SemaphoreType.DMA,
        ],
    )
    return k(xs, slot)


def _ffn_body(x_ref, w1_ref, w2_ref, out_ref, acc_ref):
    e = pl.program_id(0)
    f = pl.program_id(1)

    @pl.when(e < E)
    def _():
        ai = lax.bitcast_convert_type(x_ref[0], jnp.int32)
        xl = lax.bitcast_convert_type(ai << 16, jnp.float32)
        xh = lax.bitcast_convert_type(ai & jnp.int32(-65536), jnp.float32)
        xl = xl.astype(jnp.bfloat16)
        xh = xh.astype(jnp.bfloat16)
        w1 = w1_ref[0].astype(jnp.bfloat16)
        h = lax.dot_general(xl, w1[:, :D // 2], (((1,), (1,)), ((), ())),
                            preferred_element_type=jnp.float32)
        h = h + lax.dot_general(xh, w1[:, D // 2:], (((1,), (1,)), ((), ())),
                                preferred_element_type=jnp.float32)
        h = jnp.maximum(h, 0.0).astype(jnp.bfloat16)
        w2 = w2_ref[0].astype(jnp.bfloat16)
        part = lax.dot_general(h, w2, (((1,), (1,)), ((), ())),
                               preferred_element_type=jnp.float32)

        @pl.when(f == 0)
        def _():
            acc_ref[...] = part

        @pl.when(f > 0)
        def _():
            acc_ref[...] = acc_ref[...] + part

        @pl.when(f == NFFB - 1)
        def _():
            out_ref[0] = acc_ref[...]

    @pl.when(e == E)
    def _():
        out_ref[0] = jnp.zeros_like(out_ref[0])


def _ffn(xe, W1, W2):
    xe3 = xe.reshape(E + 1, CAP, D // 2)
    clamp_e = lambda e: jnp.minimum(e, E - 1)
    wf = lambda e, f: jnp.where(e < E, f, NFFB - 1)
    out = pl.pallas_call(
        _ffn_body,
        grid=(E + 1, NFFB),
        in_specs=[
            pl.BlockSpec((1, CAP, D // 2), lambda e, f: (clamp_e(e), 0, 0)),
            pl.BlockSpec((1, FFB, D), lambda e, f: (clamp_e(e), wf(e, f), 0)),
            pl.BlockSpec((1, D, FFB), lambda e, f: (clamp_e(e), 0, wf(e, f))),
        ],
        out_specs=pl.BlockSpec((1, CAP, D), lambda e, f: (e, 0, 0)),
        out_shape=jax.ShapeDtypeStruct((E + 1, CAP, D), jnp.float32),
        scratch_shapes=[pltpu.VMEM((CAP, D), jnp.float32)],
    )(xe3, W1, W2)
    return out.reshape((E + 1) * CAP, D)


def _collect_body(res_hbm, slot_hbm, y_hbm, slot_v, rows_v,
                  gs0, gs1, os0, os1):
    wid = lax.axis_index("s") * NC + lax.axis_index("c")
    base = wid * RPT

    pltpu.sync_copy(slot_hbm.at[pl.ds(base, RPT)], slot_v)

    _pipelined_gather(res_hbm, slot_v, y_hbm, base,
                      rows_v, (gs0, gs1), (os0, os1))


def _collect(result, slot):
    mesh = plsc.VectorSubcoreMesh(core_axis_name="c", subcore_axis_name="s")
    k = pl.kernel(
        _collect_body,
        compiler_params=pltpu.CompilerParams(needs_layout_passes=False),
        out_type=jax.ShapeDtypeStruct((T, D), jnp.float32),
        mesh=mesh,
        scratch_types=[
            pltpu.VMEM((RPT,), jnp.int32),
            pltpu.VMEM((2, GCH, D), jnp.float32),
            pltpu.SemaphoreType.DMA,
            pltpu.SemaphoreType.DMA,
            pltpu.SemaphoreType.DMA,
            pltpu.SemaphoreType.DMA,
        ],
    )
    return k(result, slot)


@jax.jit
def kernel(x, Wr, W1, W2):
    x_flat = x.reshape(T, D)
    slot, xs = _router(x_flat, Wr)
    xe = _dispatch(xs, slot)
    result = _ffn(xe, W1, W2)
    y = _collect(result, slot)
    return y.reshape(B, S, D)
